# Initial kernel scaffold; baseline (speedup 1.0000x reference)
import jax
import jax.numpy as jnp
from jax.experimental import pallas as pl


def kernel(x, z, edge_index, edge_attr, z_table, W_self, W_nbr, W_edge, b):
    raise NotImplementedError("write your pallas kernel here")


# speedup vs baseline: 4.0665x; 4.0665x over previous
import functools

import jax
import jax.numpy as jnp
from jax import lax
from jax.experimental import pallas as pl
from jax.experimental.pallas import tpu as pltpu
from jax.experimental.pallas import tpu_sc as plsc

N = 10000
E = 320000
D = 128
DE = 16
ZV = 100

C = 128
NW = 32
CPW = 79
EPAD = NW * CPW * C - E
NP = 10240
RPT = NP // 16


def _tc1_body(x_ref, z_ref, zt_ref, wn_ref, ws_ref, y_ref, hs_ref,
              tn_ref, ts_ref):
    @pl.when(pl.program_id(0) == 0)
    def _():
        tn_ref[...] = jnp.dot(zt_ref[...], wn_ref[D:, :],
                              preferred_element_type=jnp.float32)
        ts_ref[...] = jnp.dot(zt_ref[...], ws_ref[D:, :],
                              preferred_element_type=jnp.float32)

    oh = (z_ref[...] == lax.broadcasted_iota(jnp.int32, (1, D), 1)
          ).astype(jnp.float32)
    x = x_ref[...]
    y_ref[...] = (jnp.dot(x, wn_ref[:D, :], preferred_element_type=jnp.float32)
                  + jnp.dot(oh, tn_ref[...], preferred_element_type=jnp.float32))
    hs_ref[...] = (jnp.dot(x, ws_ref[:D, :], preferred_element_type=jnp.float32)
                   + jnp.dot(oh, ts_ref[...], preferred_element_type=jnp.float32))


def _tc1(x, z2, ztp, W_nbr, W_self):
    B = 1000
    return pl.pallas_call(
        _tc1_body,
        grid=(N // B,),
        in_specs=[
            pl.BlockSpec((B, D), lambda i: (i, 0)),
            pl.BlockSpec((B, 1), lambda i: (i, 0)),
            pl.BlockSpec((D, D), lambda i: (0, 0)),
            pl.BlockSpec((2 * D, D), lambda i: (0, 0)),
            pl.BlockSpec((2 * D, D), lambda i: (0, 0)),
        ],
        out_specs=[
            pl.BlockSpec((B, D), lambda i: (i, 0)),
            pl.BlockSpec((B, D), lambda i: (i, 0)),
        ],
        out_shape=[
            jax.ShapeDtypeStruct((N, D), jnp.float32),
            jax.ShapeDtypeStruct((N, D), jnp.float32),
        ],
        scratch_shapes=[
            pltpu.VMEM((D, D), jnp.float32),
            pltpu.VMEM((D, D), jnp.float32),
        ],
    )(x, z2, ztp, W_nbr, W_self)


def _sca_body(y_h, src_h, dst_h, zy_h, outy_h,
              src_v, dstbuf, ybuf, accy):
    c = lax.axis_index("c")
    s = lax.axis_index("s")
    w = s * 2 + c

    pltpu.sync_copy(zy_h, accy.at[pl.ds(s * RPT, RPT)])

    pltpu.sync_copy(src_h.at[w], src_v)

    plsc.subcore_barrier()

    def _loop(k, carry):
        pltpu.sync_copy(y_h.at[src_v.at[k]], ybuf)
        pltpu.sync_copy(dst_h.at[pl.ds((w * CPW + k) * C, C)], dstbuf)
        pltpu.sync_copy(ybuf, accy.at[dstbuf], add=True)
        return carry
    lax.fori_loop(0, CPW, _loop, 0)

    plsc.subcore_barrier()

    pltpu.sync_copy(accy.at[pl.ds(s * RPT, RPT)],
                    outy_h.at[c, pl.ds(s * RPT, RPT)])


def _sc_ysum(y, src3d, dst1d, zy):
    mesh = plsc.VectorSubcoreMesh(core_axis_name="c", subcore_axis_name="s")
    return pl.kernel(
        _sca_body,
        out_type=jax.ShapeDtypeStruct((2, NP, D), jnp.float32),
        mesh=mesh,
        scratch_types=[
            pltpu.VMEM((CPW, C), jnp.int32),
            pltpu.VMEM((C,), jnp.int32),
            pltpu.VMEM((C, D), jnp.float32),
            pltpu.VMEM_SHARED((NP, D), jnp.float32),
        ],
    )(y, src3d, dst1d, zy)


def _scb_body(dst_h, ea_h, init_h, zy_h, outa_h,
              dstbuf, eabuf, aug, acca):
    c = lax.axis_index("c")
    s = lax.axis_index("s")
    w = s * 2 + c

    pltpu.sync_copy(zy_h, acca.at[pl.ds(s * RPT, RPT)])
    pltpu.sync_copy(init_h, aug)

    plsc.subcore_barrier()

    def _loop(k, carry):
        pltpu.sync_copy(ea_h.at[pl.ds((w * CPW + k) * DE, DE)], eabuf)
        pltpu.sync_copy(dst_h.at[pl.ds((w * CPW + k) * C, C)], dstbuf)
        for q in range(DE):
            for r in range(8):
                aug[q * 8 + r, pl.ds(0, 16)] = eabuf[q, pl.ds(r * 16, 16)]
        pltpu.sync_copy(aug, acca.at[dstbuf], add=True)
        return carry
    lax.fori_loop(0, CPW, _loop, 0)

    plsc.subcore_barrier()

    pltpu.sync_copy(acca.at[pl.ds(s * RPT, RPT)],
                    outa_h.at[c, pl.ds(s * RPT, RPT)])


def _sc_easum(dst1d, eap, init, zy):
    mesh = plsc.VectorSubcoreMesh(core_axis_name="c", subcore_axis_name="s")
    return pl.kernel(
        _scb_body,
        out_type=jax.ShapeDtypeStruct((2, NP, D), jnp.float32),
        mesh=mesh,
        scratch_types=[
            pltpu.VMEM((C,), jnp.int32),
            pltpu.VMEM((DE, C), jnp.float32),
            pltpu.VMEM((C, D), jnp.float32),
            pltpu.VMEM_SHARED((NP, D), jnp.float32),
        ],
    )(dst1d, eap, init, zy)


def _tc2_body(hs_ref, ay_ref, aa_ref, we_ref, wn_ref, b_ref, o_ref):
    w2 = jnp.dot(we_ref[...], wn_ref[...], preferred_element_type=jnp.float32)
    sy = ay_ref[0] + ay_ref[1]
    aug = aa_ref[0] + aa_ref[1]
    se = aug[:, :DE]
    cnt = aug[:, DE:DE + 1]
    inv = 1.0 / jnp.maximum(cnt, 1.0)
    h = (hs_ref[...]
         + (sy + jnp.dot(se, w2, preferred_element_type=jnp.float32)) * inv
         + b_ref[...])
    o_ref[...] = jnp.where(h > 0, h, jnp.exp(h) - 1.0)


def _tc2(hs, ay, aa, W_edge, W_nbr, b2):
    B = 1000
    return pl.pallas_call(
        _tc2_body,
        grid=(N // B,),
        in_specs=[
            pl.BlockSpec((B, D), lambda i: (i, 0)),
            pl.BlockSpec((2, B, D), lambda i: (0, i, 0)),
            pl.BlockSpec((2, B, D), lambda i: (0, i, 0)),
            pl.BlockSpec((DE, 2 * D), lambda i: (0, 0)),
            pl.BlockSpec((2 * D, D), lambda i: (0, 0)),
            pl.BlockSpec((1, D), lambda i: (0, 0)),
        ],
        out_specs=pl.BlockSpec((B, D), lambda i: (i, 0)),
        out_shape=jax.ShapeDtypeStruct((N, D), jnp.float32),
    )(hs, ay, aa, W_edge, W_nbr, b2)


def kernel(x, z, edge_index, edge_attr, z_table, W_self, W_nbr, W_edge, b):
    z2 = z.astype(jnp.int32).reshape(N, 1)
    src = edge_index[0].astype(jnp.int32)
    dst = edge_index[1].astype(jnp.int32)
    src3d = jnp.concatenate([src, jnp.zeros(EPAD, jnp.int32)]
                            ).reshape(NW, CPW, C)
    dst1d = jnp.concatenate([dst, jnp.full(EPAD, N, jnp.int32)])
    eap = jnp.concatenate([edge_attr, jnp.zeros((EPAD, DE), jnp.float32)]
                          ).reshape(-1, C)
    ztp = jnp.zeros((D, D), jnp.float32).at[:ZV].set(z_table)

    y, hs = _tc1(x, z2, ztp, W_nbr, W_self)

    zy = jnp.zeros((RPT, D), jnp.float32)
    init = jnp.zeros((C, D), jnp.float32).at[:, DE].set(1.0)
    aa = _sc_easum(dst1d, eap, init, zy)
    ay = _sc_ysum(y, src3d, dst1d, zy)

    return _tc2(hs, ay, aa, W_edge, W_nbr, b.reshape(1, D))


# speedup vs baseline: 4.1697x; 1.0254x over previous
import functools

import jax
import jax.numpy as jnp
from jax import lax
from jax.experimental import pallas as pl
from jax.experimental.pallas import tpu as pltpu
from jax.experimental.pallas import tpu_sc as plsc

N = 10000
E = 320000
D = 128
DE = 16
ZV = 100

C = 128
NW = 32
CPW = 80
EPAD = NW * CPW * C - E
NP = 10240
RPT = NP // 16


def _tc1_body(x_ref, z_ref, zt_ref, wn_ref, ws_ref, y_ref, hs_ref,
              tn_ref, ts_ref):
    @pl.when(pl.program_id(0) == 0)
    def _():
        tn_ref[...] = jnp.dot(zt_ref[...], wn_ref[D:, :],
                              preferred_element_type=jnp.float32)
        ts_ref[...] = jnp.dot(zt_ref[...], ws_ref[D:, :],
                              preferred_element_type=jnp.float32)

    oh = (z_ref[...] == lax.broadcasted_iota(jnp.int32, (1, D), 1)
          ).astype(jnp.float32)
    x = x_ref[...]
    y_ref[...] = (jnp.dot(x, wn_ref[:D, :], preferred_element_type=jnp.float32)
                  + jnp.dot(oh, tn_ref[...], preferred_element_type=jnp.float32))
    hs_ref[...] = (jnp.dot(x, ws_ref[:D, :], preferred_element_type=jnp.float32)
                   + jnp.dot(oh, ts_ref[...], preferred_element_type=jnp.float32))


def _tc1(x, z2, ztp, W_nbr, W_self):
    B = 1000
    return pl.pallas_call(
        _tc1_body,
        grid=(N // B,),
        in_specs=[
            pl.BlockSpec((B, D), lambda i: (i, 0)),
            pl.BlockSpec((B, 1), lambda i: (i, 0)),
            pl.BlockSpec((D, D), lambda i: (0, 0)),
            pl.BlockSpec((2 * D, D), lambda i: (0, 0)),
            pl.BlockSpec((2 * D, D), lambda i: (0, 0)),
        ],
        out_specs=[
            pl.BlockSpec((B, D), lambda i: (i, 0)),
            pl.BlockSpec((B, D), lambda i: (i, 0)),
        ],
        out_shape=[
            jax.ShapeDtypeStruct((N, D), jnp.float32),
            jax.ShapeDtypeStruct((N, D), jnp.float32),
        ],
        scratch_shapes=[
            pltpu.VMEM((D, D), jnp.float32),
            pltpu.VMEM((D, D), jnp.float32),
        ],
    )(x, z2, ztp, W_nbr, W_self)


NBUF = 2


def _sca_body(y_h, src_h, dst_h, zy_h, outy_h,
              src_v, dbufs, ybufs, accy, gsems, dsems, ssems):
    c = lax.axis_index("c")
    s = lax.axis_index("s")
    w = s * 2 + c

    pltpu.sync_copy(zy_h, accy.at[pl.ds(s * RPT, RPT)])

    pltpu.sync_copy(src_h.at[w], src_v)

    def _start(k, b):
        pltpu.make_async_copy(y_h.at[src_v.at[k]], ybufs[b], gsems[b]).start()
        pltpu.make_async_copy(dst_h.at[pl.ds((w * CPW + k) * C, C)],
                              dbufs[b], dsems[b]).start()

    plsc.subcore_barrier()

    _start(0, 0)

    def _loop(j, carry):
        for b in range(NBUF):
            k = NBUF * j + b

            @pl.when(k >= NBUF - 1)
            def _():
                pltpu.make_async_copy(ybufs[(b + 1) % NBUF],
                                      accy.at[dbufs[(b + 1) % NBUF]],
                                      ssems[(b + 1) % NBUF]).wait()

            @pl.when(k + 1 < CPW)
            def _():
                _start(k + 1, (b + 1) % NBUF)

            pltpu.make_async_copy(y_h.at[src_v.at[k]], ybufs[b],
                                  gsems[b]).wait()
            pltpu.make_async_copy(dst_h.at[pl.ds(0, C)], dbufs[b],
                                  dsems[b]).wait()
            pltpu.make_async_copy(ybufs[b], accy.at[dbufs[b]],
                                  ssems[b]).start(add=True)
        return carry
    lax.fori_loop(0, CPW // NBUF, _loop, 0)

    for t in range(1, NBUF):
        pltpu.make_async_copy(ybufs[t], accy.at[dbufs[t]], ssems[t]).wait()

    plsc.subcore_barrier()

    pltpu.sync_copy(accy.at[pl.ds(s * RPT, RPT)],
                    outy_h.at[c, pl.ds(s * RPT, RPT)])


def _sc_ysum(y, src3d, dst1d, zy):
    mesh = plsc.VectorSubcoreMesh(core_axis_name="c", subcore_axis_name="s")
    return pl.kernel(
        _sca_body,
        out_type=jax.ShapeDtypeStruct((2, NP, D), jnp.float32),
        mesh=mesh,
        scratch_types=[
            pltpu.VMEM((CPW, C), jnp.int32),
            [pltpu.VMEM((C,), jnp.int32) for _ in range(NBUF)],
            [pltpu.VMEM((C, D), jnp.float32) for _ in range(NBUF)],
            pltpu.VMEM_SHARED((NP, D), jnp.float32),
            [pltpu.SemaphoreType.DMA for _ in range(NBUF)],
            [pltpu.SemaphoreType.DMA for _ in range(NBUF)],
            [pltpu.SemaphoreType.DMA for _ in range(NBUF)],
        ],
    )(y, src3d, dst1d, zy)


def _scb_body(dst_h, ea_h, init_h, zy_h, outa_h,
              dbufs, eabufs, augs, acca, esems, dsems, ssems):
    c = lax.axis_index("c")
    s = lax.axis_index("s")
    w = s * 2 + c

    pltpu.sync_copy(zy_h, acca.at[pl.ds(s * RPT, RPT)])
    for b in range(NBUF):
        pltpu.sync_copy(init_h, augs[b])

    def _start(k, b):
        pltpu.make_async_copy(ea_h.at[pl.ds((w * CPW + k) * DE, DE)],
                              eabufs[b], esems[b]).start()
        pltpu.make_async_copy(dst_h.at[pl.ds((w * CPW + k) * C, C)],
                              dbufs[b], dsems[b]).start()

    plsc.subcore_barrier()

    _start(0, 0)

    def _loop(j, carry):
        for b in range(NBUF):
            k = NBUF * j + b

            @pl.when(k >= NBUF - 1)
            def _():
                pltpu.make_async_copy(augs[(b + 1) % NBUF],
                                      acca.at[dbufs[(b + 1) % NBUF]],
                                      ssems[(b + 1) % NBUF]).wait()

            @pl.when(k + 1 < CPW)
            def _():
                _start(k + 1, (b + 1) % NBUF)

            pltpu.make_async_copy(ea_h.at[pl.ds(0, DE)], eabufs[b],
                                  esems[b]).wait()
            pltpu.make_async_copy(dst_h.at[pl.ds(0, C)], dbufs[b],
                                  dsems[b]).wait()
            for q in range(DE):
                for r in range(8):
                    augs[b][q * 8 + r, pl.ds(0, 16)] = \
                        eabufs[b][q, pl.ds(r * 16, 16)]
            pltpu.make_async_copy(augs[b], acca.at[dbufs[b]],
                                  ssems[b]).start(add=True)
        return carry
    lax.fori_loop(0, CPW // NBUF, _loop, 0)

    for t in range(1, NBUF):
        pltpu.make_async_copy(augs[t], acca.at[dbufs[t]], ssems[t]).wait()

    plsc.subcore_barrier()

    pltpu.sync_copy(acca.at[pl.ds(s * RPT, RPT)],
                    outa_h.at[c, pl.ds(s * RPT, RPT)])


def _sc_easum(dst1d, eap, init, zy):
    mesh = plsc.VectorSubcoreMesh(core_axis_name="c", subcore_axis_name="s")
    return pl.kernel(
        _scb_body,
        out_type=jax.ShapeDtypeStruct((2, NP, D), jnp.float32),
        mesh=mesh,
        scratch_types=[
            [pltpu.VMEM((C,), jnp.int32) for _ in range(NBUF)],
            [pltpu.VMEM((DE, C), jnp.float32) for _ in range(NBUF)],
            [pltpu.VMEM((C, D), jnp.float32) for _ in range(NBUF)],
            pltpu.VMEM_SHARED((NP, D), jnp.float32),
            [pltpu.SemaphoreType.DMA for _ in range(NBUF)],
            [pltpu.SemaphoreType.DMA for _ in range(NBUF)],
            [pltpu.SemaphoreType.DMA for _ in range(NBUF)],
        ],
    )(dst1d, eap, init, zy)


def _tc2_body(hs_ref, ay_ref, aa_ref, we_ref, wn_ref, b_ref, o_ref):
    w2 = jnp.dot(we_ref[...], wn_ref[...], preferred_element_type=jnp.float32)
    sy = ay_ref[0] + ay_ref[1]
    aug = aa_ref[0] + aa_ref[1]
    se = aug[:, :DE]
    cnt = aug[:, DE:DE + 1]
    inv = 1.0 / jnp.maximum(cnt, 1.0)
    h = (hs_ref[...]
         + (sy + jnp.dot(se, w2, preferred_element_type=jnp.float32)) * inv
         + b_ref[...])
    o_ref[...] = jnp.where(h > 0, h, jnp.exp(h) - 1.0)


def _tc2(hs, ay, aa, W_edge, W_nbr, b2):
    B = 1000
    return pl.pallas_call(
        _tc2_body,
        grid=(N // B,),
        in_specs=[
            pl.BlockSpec((B, D), lambda i: (i, 0)),
            pl.BlockSpec((2, B, D), lambda i: (0, i, 0)),
            pl.BlockSpec((2, B, D), lambda i: (0, i, 0)),
            pl.BlockSpec((DE, 2 * D), lambda i: (0, 0)),
            pl.BlockSpec((2 * D, D), lambda i: (0, 0)),
            pl.BlockSpec((1, D), lambda i: (0, 0)),
        ],
        out_specs=pl.BlockSpec((B, D), lambda i: (i, 0)),
        out_shape=jax.ShapeDtypeStruct((N, D), jnp.float32),
    )(hs, ay, aa, W_edge, W_nbr, b2)


def kernel(x, z, edge_index, edge_attr, z_table, W_self, W_nbr, W_edge, b):
    z2 = z.astype(jnp.int32).reshape(N, 1)
    src = edge_index[0].astype(jnp.int32)
    dst = edge_index[1].astype(jnp.int32)
    src3d = jnp.concatenate([src, jnp.zeros(EPAD, jnp.int32)]
                            ).reshape(NW, CPW, C)
    dst1d = jnp.concatenate([dst, jnp.full(EPAD, N, jnp.int32)])
    eap = jnp.concatenate([edge_attr, jnp.zeros((EPAD, DE), jnp.float32)]
                          ).reshape(-1, C)
    ztp = jnp.zeros((D, D), jnp.float32).at[:ZV].set(z_table)

    y, hs = _tc1(x, z2, ztp, W_nbr, W_self)

    zy = jnp.zeros((RPT, D), jnp.float32)
    init = jnp.zeros((C, D), jnp.float32).at[:, DE].set(1.0)
    aa = _sc_easum(dst1d, eap, init, zy)
    ay = _sc_ysum(y, src3d, dst1d, zy)

    return _tc2(hs, ay, aa, W_edge, W_nbr, b.reshape(1, D))


# speedup vs baseline: 9.4102x; 2.2568x over previous
import functools

import jax
import jax.numpy as jnp
from jax import lax
from jax.experimental import pallas as pl
from jax.experimental.pallas import tpu as pltpu
from jax.experimental.pallas import tpu_sc as plsc

N = 10000
E = 320000
D = 128
DE = 16
ZV = 100

C = 128
NW = 32
NCHUNK = E // C
CPT = NCHUNK // NW
REM = NCHUNK - CPT * NW
NP = 10240
RPT = NP // 16
NBUF = 2


def _tc1_body(x_ref, z_ref, zt_ref, wn_ref, ws_ref, y_ref, hs_ref,
              tn_ref, ts_ref):
    @pl.when(pl.program_id(0) == 0)
    def _():
        tn_ref[...] = jnp.dot(zt_ref[...], wn_ref[D:, :],
                              preferred_element_type=jnp.float32)
        ts_ref[...] = jnp.dot(zt_ref[...], ws_ref[D:, :],
                              preferred_element_type=jnp.float32)

    oh = (z_ref[...] == lax.broadcasted_iota(jnp.int32, (1, D), 1)
          ).astype(jnp.float32)
    x = x_ref[...]
    y_ref[...] = (jnp.dot(x, wn_ref[:D, :], preferred_element_type=jnp.float32)
                  + jnp.dot(oh, tn_ref[...], preferred_element_type=jnp.float32))
    hs_ref[...] = (jnp.dot(x, ws_ref[:D, :], preferred_element_type=jnp.float32)
                   + jnp.dot(oh, ts_ref[...], preferred_element_type=jnp.float32))


def _tc1(x, z2, ztp, W_nbr, W_self):
    B = 1000
    return pl.pallas_call(
        _tc1_body,
        grid=(N // B,),
        in_specs=[
            pl.BlockSpec((B, D), lambda i: (i, 0)),
            pl.BlockSpec((B, 1), lambda i: (i, 0)),
            pl.BlockSpec((D, D), lambda i: (0, 0)),
            pl.BlockSpec((2 * D, D), lambda i: (0, 0)),
            pl.BlockSpec((2 * D, D), lambda i: (0, 0)),
        ],
        out_specs=[
            pl.BlockSpec((B, D), lambda i: (i, 0)),
            pl.BlockSpec((B, D), lambda i: (i, 0)),
        ],
        out_shape=[
            jax.ShapeDtypeStruct((N, D), jnp.float32),
            jax.ShapeDtypeStruct((N, D), jnp.float32),
        ],
        scratch_shapes=[
            pltpu.VMEM((D, D), jnp.float32),
            pltpu.VMEM((D, D), jnp.float32),
        ],
    )(x, z2, ztp, W_nbr, W_self)


def _sca_body(y_h, src_h, dst_h, zy_h, outy_h,
              sbufs, dbufs, ybufs, accy, gsems, dsems, ssems):
    c = lax.axis_index("c")
    s = lax.axis_index("s")
    w = s * 2 + c

    pltpu.sync_copy(zy_h, accy.at[pl.ds(s * RPT, RPT)])

    def _start(k, b):
        ch = w + NW * k
        pltpu.sync_copy(src_h.at[pl.ds(ch * C, C)], sbufs[b])
        pltpu.make_async_copy(y_h.at[sbufs[b]], ybufs[b], gsems[b]).start()
        pltpu.make_async_copy(dst_h.at[pl.ds(ch * C, C)],
                              dbufs[b], dsems[b]).start()

    def _finish(b):
        pltpu.make_async_copy(y_h.at[sbufs[b]], ybufs[b], gsems[b]).wait()
        pltpu.make_async_copy(dst_h.at[pl.ds(0, C)], dbufs[b],
                              dsems[b]).wait()
        pltpu.make_async_copy(ybufs[b], accy.at[dbufs[b]],
                              ssems[b]).start(add=True)

    plsc.subcore_barrier()

    _start(0, 0)

    def _loop(j, carry):
        for b in range(NBUF):
            k = NBUF * j + b

            @pl.when(k >= NBUF - 1)
            def _():
                pltpu.make_async_copy(ybufs[(b + 1) % NBUF],
                                      accy.at[dbufs[(b + 1) % NBUF]],
                                      ssems[(b + 1) % NBUF]).wait()

            @pl.when(k + 1 < CPT)
            def _():
                _start(k + 1, (b + 1) % NBUF)

            _finish(b)
        return carry
    lax.fori_loop(0, CPT // NBUF, _loop, 0)

    for t in range(1, NBUF):
        pltpu.make_async_copy(ybufs[t], accy.at[dbufs[t]], ssems[t]).wait()

    @pl.when(w < REM)
    def _():
        _start(CPT, 0)
        _finish(0)
        pltpu.make_async_copy(ybufs[0], accy.at[dbufs[0]], ssems[0]).wait()

    plsc.subcore_barrier()

    pltpu.sync_copy(accy.at[pl.ds(s * RPT, RPT)],
                    outy_h.at[c, pl.ds(s * RPT, RPT)])


def _sc_ysum(y, src1d, dst1d, zy):
    mesh = plsc.VectorSubcoreMesh(core_axis_name="c", subcore_axis_name="s")
    return pl.kernel(
        _sca_body,
        out_type=jax.ShapeDtypeStruct((2, NP, D), jnp.float32),
        mesh=mesh,
        scratch_types=[
            [pltpu.VMEM((C,), jnp.int32) for _ in range(NBUF)],
            [pltpu.VMEM((C,), jnp.int32) for _ in range(NBUF)],
            [pltpu.VMEM((C, D), jnp.float32) for _ in range(NBUF)],
            pltpu.VMEM_SHARED((NP, D), jnp.float32),
            [pltpu.SemaphoreType.DMA for _ in range(NBUF)],
            [pltpu.SemaphoreType.DMA for _ in range(NBUF)],
            [pltpu.SemaphoreType.DMA for _ in range(NBUF)],
        ],
    )(y, src1d, dst1d, zy)


def _scb_body(dst_h, ea_h, init_h, zy_h, outa_h,
              dbufs, eabufs, augs, acca, esems, dsems, ssems):
    c = lax.axis_index("c")
    s = lax.axis_index("s")
    w = s * 2 + c

    pltpu.sync_copy(zy_h, acca.at[pl.ds(s * RPT, RPT)])
    for b in range(NBUF):
        pltpu.sync_copy(init_h, augs[b])

    def _start(k, b):
        ch = w + NW * k
        pltpu.make_async_copy(ea_h.at[pl.ds(ch * DE, DE)],
                              eabufs[b], esems[b]).start()
        pltpu.make_async_copy(dst_h.at[pl.ds(ch * C, C)],
                              dbufs[b], dsems[b]).start()

    def _finish(b):
        pltpu.make_async_copy(ea_h.at[pl.ds(0, DE)], eabufs[b],
                              esems[b]).wait()
        pltpu.make_async_copy(dst_h.at[pl.ds(0, C)], dbufs[b],
                              dsems[b]).wait()
        for q in range(DE):
            for r in range(8):
                augs[b][q * 8 + r, pl.ds(0, 16)] = \
                    eabufs[b][q, pl.ds(r * 16, 16)]
        pltpu.make_async_copy(augs[b], acca.at[dbufs[b]],
                              ssems[b]).start(add=True)

    plsc.subcore_barrier()

    _start(0, 0)

    def _loop(j, carry):
        for b in range(NBUF):
            k = NBUF * j + b

            @pl.when(k >= NBUF - 1)
            def _():
                pltpu.make_async_copy(augs[(b + 1) % NBUF],
                                      acca.at[dbufs[(b + 1) % NBUF]],
                                      ssems[(b + 1) % NBUF]).wait()

            @pl.when(k + 1 < CPT)
            def _():
                _start(k + 1, (b + 1) % NBUF)

            _finish(b)
        return carry
    lax.fori_loop(0, CPT // NBUF, _loop, 0)

    for t in range(1, NBUF):
        pltpu.make_async_copy(augs[t], acca.at[dbufs[t]], ssems[t]).wait()

    @pl.when(w < REM)
    def _():
        _start(CPT, 0)
        _finish(0)
        pltpu.make_async_copy(augs[0], acca.at[dbufs[0]], ssems[0]).wait()

    plsc.subcore_barrier()

    pltpu.sync_copy(acca.at[pl.ds(s * RPT, RPT)],
                    outa_h.at[c, pl.ds(s * RPT, RPT)])


def _sc_easum(dst1d, eap, init, zy):
    mesh = plsc.VectorSubcoreMesh(core_axis_name="c", subcore_axis_name="s")
    return pl.kernel(
        _scb_body,
        out_type=jax.ShapeDtypeStruct((2, NP, D), jnp.float32),
        mesh=mesh,
        scratch_types=[
            [pltpu.VMEM((C,), jnp.int32) for _ in range(NBUF)],
            [pltpu.VMEM((DE, C), jnp.float32) for _ in range(NBUF)],
            [pltpu.VMEM((C, D), jnp.float32) for _ in range(NBUF)],
            pltpu.VMEM_SHARED((NP, D), jnp.float32),
            [pltpu.SemaphoreType.DMA for _ in range(NBUF)],
            [pltpu.SemaphoreType.DMA for _ in range(NBUF)],
            [pltpu.SemaphoreType.DMA for _ in range(NBUF)],
        ],
    )(dst1d, eap, init, zy)


def _tc2_body(hs_ref, ay_ref, aa_ref, we_ref, wn_ref, b_ref, o_ref):
    w2 = jnp.dot(we_ref[...], wn_ref[...], preferred_element_type=jnp.float32)
    sy = ay_ref[0] + ay_ref[1]
    aug = aa_ref[0] + aa_ref[1]
    se = aug[:, :DE]
    cnt = aug[:, DE:DE + 1]
    inv = 1.0 / jnp.maximum(cnt, 1.0)
    h = (hs_ref[...]
         + (sy + jnp.dot(se, w2, preferred_element_type=jnp.float32)) * inv
         + b_ref[...])
    o_ref[...] = jnp.where(h > 0, h, jnp.exp(h) - 1.0)


def _tc2(hs, ay, aa, W_edge, W_nbr, b2):
    B = 1000
    return pl.pallas_call(
        _tc2_body,
        grid=(N // B,),
        in_specs=[
            pl.BlockSpec((B, D), lambda i: (i, 0)),
            pl.BlockSpec((2, B, D), lambda i: (0, i, 0)),
            pl.BlockSpec((2, B, D), lambda i: (0, i, 0)),
            pl.BlockSpec((DE, 2 * D), lambda i: (0, 0)),
            pl.BlockSpec((2 * D, D), lambda i: (0, 0)),
            pl.BlockSpec((1, D), lambda i: (0, 0)),
        ],
        out_specs=pl.BlockSpec((B, D), lambda i: (i, 0)),
        out_shape=jax.ShapeDtypeStruct((N, D), jnp.float32),
    )(hs, ay, aa, W_edge, W_nbr, b2)


def kernel(x, z, edge_index, edge_attr, z_table, W_self, W_nbr, W_edge, b):
    z2 = z.astype(jnp.int32).reshape(N, 1)
    src1d = edge_index[0].astype(jnp.int32)
    dst1d = edge_index[1].astype(jnp.int32)
    eap = edge_attr.reshape(E // 8, C)
    ztp = jnp.zeros((D, D), jnp.float32).at[:ZV].set(z_table)

    y, hs = _tc1(x, z2, ztp, W_nbr, W_self)

    zy = jnp.zeros((RPT, D), jnp.float32)
    init = jnp.zeros((C, D), jnp.float32).at[:, DE].set(1.0)
    aa = _sc_easum(dst1d, eap, init, zy)
    ay = _sc_ysum(y, src1d, dst1d, zy)

    return _tc2(hs, ay, aa, W_edge, W_nbr, b.reshape(1, D))


# speedup vs baseline: 10.2436x; 1.0886x over previous
import functools

import jax
import jax.numpy as jnp
from jax import lax
from jax.experimental import pallas as pl
from jax.experimental.pallas import tpu as pltpu
from jax.experimental.pallas import tpu_sc as plsc

N = 10000
E = 320000
D = 128
DE = 16
ZV = 100

C = 128
NW = 32
NCHUNK = E // C
CPT = NCHUNK // NW
REM = NCHUNK - CPT * NW
NP = 10112
RPT = NP // 16
NBUFA = 3
NBUFB = 2


def _tc1_body(x_ref, z_ref, zt_ref, wn_ref, ws_ref, y_ref, hs_ref,
              tn_ref, ts_ref):
    @pl.when(pl.program_id(0) == 0)
    def _():
        tn_ref[...] = jnp.dot(zt_ref[...], wn_ref[D:, :],
                              preferred_element_type=jnp.float32)
        ts_ref[...] = jnp.dot(zt_ref[...], ws_ref[D:, :],
                              preferred_element_type=jnp.float32)

    oh = (z_ref[...] == lax.broadcasted_iota(jnp.int32, (1, D), 1)
          ).astype(jnp.float32)
    x = x_ref[...]
    y_ref[...] = (jnp.dot(x, wn_ref[:D, :], preferred_element_type=jnp.float32)
                  + jnp.dot(oh, tn_ref[...], preferred_element_type=jnp.float32))
    hs_ref[...] = (jnp.dot(x, ws_ref[:D, :], preferred_element_type=jnp.float32)
                   + jnp.dot(oh, ts_ref[...], preferred_element_type=jnp.float32))


def _tc1(x, z2, ztp, W_nbr, W_self):
    B = 1000
    return pl.pallas_call(
        _tc1_body,
        grid=(N // B,),
        in_specs=[
            pl.BlockSpec((B, D), lambda i: (i, 0)),
            pl.BlockSpec((B, 1), lambda i: (i, 0)),
            pl.BlockSpec((D, D), lambda i: (0, 0)),
            pl.BlockSpec((2 * D, D), lambda i: (0, 0)),
            pl.BlockSpec((2 * D, D), lambda i: (0, 0)),
        ],
        out_specs=[
            pl.BlockSpec((B, D), lambda i: (i, 0)),
            pl.BlockSpec((B, D), lambda i: (i, 0)),
        ],
        out_shape=[
            jax.ShapeDtypeStruct((N, D), jnp.float32),
            jax.ShapeDtypeStruct((N, D), jnp.float32),
        ],
        scratch_shapes=[
            pltpu.VMEM((D, D), jnp.float32),
            pltpu.VMEM((D, D), jnp.float32),
        ],
    )(x, z2, ztp, W_nbr, W_self)


def _sca_body(y_h, ei_h, zy_h, outy_h,
              sbufs, dbufs, ybufs, accy, gsems, dsems, ssems):
    c = lax.axis_index("c")
    s = lax.axis_index("s")
    w = s * 2 + c

    pltpu.sync_copy(zy_h, accy.at[pl.ds(s * RPT, RPT)])

    def _start(k, b):
        ch = w + NW * k
        pltpu.sync_copy(ei_h.at[0, pl.ds(ch * C, C)], sbufs[b])
        pltpu.make_async_copy(y_h.at[sbufs[b]], ybufs[b], gsems[b]).start()
        pltpu.make_async_copy(ei_h.at[1, pl.ds(ch * C, C)],
                              dbufs[b], dsems[b]).start()

    def _finish(b):
        pltpu.make_async_copy(y_h.at[sbufs[b]], ybufs[b], gsems[b]).wait()
        pltpu.make_async_copy(ei_h.at[1, pl.ds(0, C)], dbufs[b],
                              dsems[b]).wait()
        pltpu.make_async_copy(ybufs[b], accy.at[dbufs[b]],
                              ssems[b]).start(add=True)

    plsc.subcore_barrier()

    _start(0, 0)

    def _loop(j, carry):
        for b in range(NBUFA):
            k = NBUFA * j + b

            @pl.when(k >= NBUFA - 1)
            def _():
                pltpu.make_async_copy(ybufs[(b + 1) % NBUFA],
                                      accy.at[dbufs[(b + 1) % NBUFA]],
                                      ssems[(b + 1) % NBUFA]).wait()

            @pl.when(k + 1 < CPT)
            def _():
                _start(k + 1, (b + 1) % NBUFA)

            _finish(b)
        return carry
    lax.fori_loop(0, CPT // NBUFA, _loop, 0)

    for t in range(1, NBUFA):
        pltpu.make_async_copy(ybufs[t], accy.at[dbufs[t]], ssems[t]).wait()

    @pl.when(w < REM)
    def _():
        _start(CPT, 0)
        _finish(0)
        pltpu.make_async_copy(ybufs[0], accy.at[dbufs[0]], ssems[0]).wait()

    plsc.subcore_barrier()

    pltpu.sync_copy(accy.at[pl.ds(s * RPT, RPT)],
                    outy_h.at[c, pl.ds(s * RPT, RPT)])


def _sc_ysum(y, ei, zy):
    mesh = plsc.VectorSubcoreMesh(core_axis_name="c", subcore_axis_name="s")
    return pl.kernel(
        _sca_body,
        out_type=jax.ShapeDtypeStruct((2, NP, D), jnp.float32),
        mesh=mesh,
        scratch_types=[
            [pltpu.VMEM((C,), jnp.int32) for _ in range(NBUFA)],
            [pltpu.VMEM((C,), jnp.int32) for _ in range(NBUFA)],
            [pltpu.VMEM((C, D), jnp.float32) for _ in range(NBUFA)],
            pltpu.VMEM_SHARED((NP, D), jnp.float32),
            [pltpu.SemaphoreType.DMA for _ in range(NBUFA)],
            [pltpu.SemaphoreType.DMA for _ in range(NBUFA)],
            [pltpu.SemaphoreType.DMA for _ in range(NBUFA)],
        ],
    )(y, ei, zy)


def _scb_body(ei_h, ea_h, init_h, zy_h, outa_h,
              dbufs, eabufs, augs, acca, esems, dsems, ssems):
    c = lax.axis_index("c")
    s = lax.axis_index("s")
    w = s * 2 + c

    pltpu.sync_copy(zy_h, acca.at[pl.ds(s * RPT, RPT)])
    for b in range(NBUFB):
        pltpu.sync_copy(init_h, augs[b])

    def _start(k, b):
        ch = w + NW * k
        pltpu.make_async_copy(ea_h.at[pl.ds(ch * DE, DE)],
                              eabufs[b], esems[b]).start()
        pltpu.make_async_copy(ei_h.at[1, pl.ds(ch * C, C)],
                              dbufs[b], dsems[b]).start()

    def _finish(b):
        pltpu.make_async_copy(ea_h.at[pl.ds(0, DE)], eabufs[b],
                              esems[b]).wait()
        pltpu.make_async_copy(ei_h.at[1, pl.ds(0, C)], dbufs[b],
                              dsems[b]).wait()
        for q in range(DE):
            for r in range(8):
                augs[b][q * 8 + r, pl.ds(0, 16)] = \
                    eabufs[b][q, pl.ds(r * 16, 16)]
        pltpu.make_async_copy(augs[b], acca.at[dbufs[b]],
                              ssems[b]).start(add=True)

    plsc.subcore_barrier()

    _start(0, 0)

    def _loop(j, carry):
        for b in range(NBUFB):
            k = NBUFB * j + b

            @pl.when(k >= NBUFB - 1)
            def _():
                pltpu.make_async_copy(augs[(b + 1) % NBUFB],
                                      acca.at[dbufs[(b + 1) % NBUFB]],
                                      ssems[(b + 1) % NBUFB]).wait()

            @pl.when(k + 1 < CPT)
            def _():
                _start(k + 1, (b + 1) % NBUFB)

            _finish(b)
        return carry
    lax.fori_loop(0, CPT // NBUFB, _loop, 0)

    for t in range(1, NBUFB):
        pltpu.make_async_copy(augs[t], acca.at[dbufs[t]], ssems[t]).wait()

    @pl.when(w < REM)
    def _():
        _start(CPT, 0)
        _finish(0)
        pltpu.make_async_copy(augs[0], acca.at[dbufs[0]], ssems[0]).wait()

    plsc.subcore_barrier()

    pltpu.sync_copy(acca.at[pl.ds(s * RPT, RPT)],
                    outa_h.at[c, pl.ds(s * RPT, RPT)])


def _sc_easum(ei, eap, init, zy):
    mesh = plsc.VectorSubcoreMesh(core_axis_name="c", subcore_axis_name="s")
    return pl.kernel(
        _scb_body,
        out_type=jax.ShapeDtypeStruct((2, NP, D), jnp.float32),
        mesh=mesh,
        scratch_types=[
            [pltpu.VMEM((C,), jnp.int32) for _ in range(NBUFB)],
            [pltpu.VMEM((DE, C), jnp.float32) for _ in range(NBUFB)],
            [pltpu.VMEM((C, D), jnp.float32) for _ in range(NBUFB)],
            pltpu.VMEM_SHARED((NP, D), jnp.float32),
            [pltpu.SemaphoreType.DMA for _ in range(NBUFB)],
            [pltpu.SemaphoreType.DMA for _ in range(NBUFB)],
            [pltpu.SemaphoreType.DMA for _ in range(NBUFB)],
        ],
    )(ei, eap, init, zy)


def _tc2_body(hs_ref, ay_ref, aa_ref, we_ref, wn_ref, b_ref, o_ref):
    w2 = jnp.dot(we_ref[...], wn_ref[...], preferred_element_type=jnp.float32)
    sy = ay_ref[0] + ay_ref[1]
    aug = aa_ref[0] + aa_ref[1]
    se = aug[:, :DE]
    cnt = aug[:, DE:DE + 1]
    inv = 1.0 / jnp.maximum(cnt, 1.0)
    h = (hs_ref[...]
         + (sy + jnp.dot(se, w2, preferred_element_type=jnp.float32)) * inv
         + b_ref[...])
    o_ref[...] = jnp.where(h > 0, h, jnp.exp(h) - 1.0)


def _tc2(hs, ay, aa, W_edge, W_nbr, b2):
    B = 1000
    return pl.pallas_call(
        _tc2_body,
        grid=(N // B,),
        in_specs=[
            pl.BlockSpec((B, D), lambda i: (i, 0)),
            pl.BlockSpec((2, B, D), lambda i: (0, i, 0)),
            pl.BlockSpec((2, B, D), lambda i: (0, i, 0)),
            pl.BlockSpec((DE, 2 * D), lambda i: (0, 0)),
            pl.BlockSpec((2 * D, D), lambda i: (0, 0)),
            pl.BlockSpec((1, D), lambda i: (0, 0)),
        ],
        out_specs=pl.BlockSpec((B, D), lambda i: (i, 0)),
        out_shape=jax.ShapeDtypeStruct((N, D), jnp.float32),
    )(hs, ay, aa, W_edge, W_nbr, b2)


def kernel(x, z, edge_index, edge_attr, z_table, W_self, W_nbr, W_edge, b):
    z2 = z.astype(jnp.int32).reshape(N, 1)
    ei = edge_index.astype(jnp.int32)
    eap = edge_attr.reshape(E // 8, C)
    ztp = jnp.zeros((D, D), jnp.float32).at[:ZV].set(z_table)

    y, hs = _tc1(x, z2, ztp, W_nbr, W_self)

    zy = jnp.zeros((RPT, D), jnp.float32)
    init = jnp.zeros((C, D), jnp.float32).at[:, DE].set(1.0)
    aa = _sc_easum(ei, eap, init, zy)
    ay = _sc_ysum(y, ei, zy)

    return _tc2(hs, ay, aa, W_edge, W_nbr, b.reshape(1, D))


# speedup vs baseline: 10.3142x; 1.0069x over previous
import functools

import jax
import jax.numpy as jnp
from jax import lax
from jax.experimental import pallas as pl
from jax.experimental.pallas import tpu as pltpu
from jax.experimental.pallas import tpu_sc as plsc

N = 10000
E = 320000
D = 128
DE = 16
ZV = 100

C = 128
NW = 32
NCHUNK = E // C
CPT = NCHUNK // NW
REM = NCHUNK - CPT * NW
NP = 10112
RPT = NP // 16
NBUFA = 3
NBUFB = 2


def _tcp_body(x_ref, z_ref, zt_ref, w_ref, o_ref, t_ref):
    @pl.when(pl.program_id(0) == 0)
    def _():
        t_ref[...] = jnp.dot(zt_ref[...], w_ref[D:, :],
                             preferred_element_type=jnp.float32)

    oh = (z_ref[...] == lax.broadcasted_iota(jnp.int32, (1, D), 1)
          ).astype(jnp.float32)
    o_ref[...] = (jnp.dot(x_ref[...], w_ref[:D, :],
                          preferred_element_type=jnp.float32)
                  + jnp.dot(oh, t_ref[...],
                            preferred_element_type=jnp.float32))


def _tc_proj(x, z2, ztp, W):
    B = 2000
    return pl.pallas_call(
        _tcp_body,
        grid=(N // B,),
        in_specs=[
            pl.BlockSpec((B, D), lambda i: (i, 0)),
            pl.BlockSpec((B, 1), lambda i: (i, 0)),
            pl.BlockSpec((D, D), lambda i: (0, 0)),
            pl.BlockSpec((2 * D, D), lambda i: (0, 0)),
        ],
        out_specs=pl.BlockSpec((B, D), lambda i: (i, 0)),
        out_shape=jax.ShapeDtypeStruct((N, D), jnp.float32),
        scratch_shapes=[
            pltpu.VMEM((D, D), jnp.float32),
        ],
    )(x, z2, ztp, W)


def _sca_body(y_h, ei_h, zy_h, outy_h,
              sbufs, dbufs, ybufs, accy, gsems, dsems, ssems):
    c = lax.axis_index("c")
    s = lax.axis_index("s")
    w = s * 2 + c

    pltpu.sync_copy(zy_h, accy.at[pl.ds(s * RPT, RPT)])

    def _start(k, b):
        ch = w + NW * k
        pltpu.sync_copy(ei_h.at[0, pl.ds(ch * C, C)], sbufs[b])
        pltpu.make_async_copy(y_h.at[sbufs[b]], ybufs[b], gsems[b]).start()
        pltpu.make_async_copy(ei_h.at[1, pl.ds(ch * C, C)],
                              dbufs[b], dsems[b]).start()

    def _finish(b):
        pltpu.make_async_copy(y_h.at[sbufs[b]], ybufs[b], gsems[b]).wait()
        pltpu.make_async_copy(ei_h.at[1, pl.ds(0, C)], dbufs[b],
                              dsems[b]).wait()
        pltpu.make_async_copy(ybufs[b], accy.at[dbufs[b]],
                              ssems[b]).start(add=True)

    plsc.subcore_barrier()

    _start(0, 0)

    def _loop(j, carry):
        for b in range(NBUFA):
            k = NBUFA * j + b

            @pl.when(k >= NBUFA - 1)
            def _():
                pltpu.make_async_copy(ybufs[(b + 1) % NBUFA],
                                      accy.at[dbufs[(b + 1) % NBUFA]],
                                      ssems[(b + 1) % NBUFA]).wait()

            @pl.when(k + 1 < CPT)
            def _():
                _start(k + 1, (b + 1) % NBUFA)

            _finish(b)
        return carry
    lax.fori_loop(0, CPT // NBUFA, _loop, 0)

    for t in range(1, NBUFA):
        pltpu.make_async_copy(ybufs[t], accy.at[dbufs[t]], ssems[t]).wait()

    @pl.when(w < REM)
    def _():
        _start(CPT, 0)
        _finish(0)
        pltpu.make_async_copy(ybufs[0], accy.at[dbufs[0]], ssems[0]).wait()

    plsc.subcore_barrier()

    pltpu.sync_copy(accy.at[pl.ds(s * RPT, RPT)],
                    outy_h.at[c, pl.ds(s * RPT, RPT)])


def _sc_ysum(y, ei, zy):
    mesh = plsc.VectorSubcoreMesh(core_axis_name="c", subcore_axis_name="s")
    return pl.kernel(
        _sca_body,
        out_type=jax.ShapeDtypeStruct((2, NP, D), jnp.float32),
        mesh=mesh,
        scratch_types=[
            [pltpu.VMEM((C,), jnp.int32) for _ in range(NBUFA)],
            [pltpu.VMEM((C,), jnp.int32) for _ in range(NBUFA)],
            [pltpu.VMEM((C, D), jnp.float32) for _ in range(NBUFA)],
            pltpu.VMEM_SHARED((NP, D), jnp.float32),
            [pltpu.SemaphoreType.DMA for _ in range(NBUFA)],
            [pltpu.SemaphoreType.DMA for _ in range(NBUFA)],
            [pltpu.SemaphoreType.DMA for _ in range(NBUFA)],
        ],
    )(y, ei, zy)


def _scb_body(ei_h, ea_h, init_h, zy_h, outa_h,
              dbufs, eabufs, augs, acca, esems, dsems, ssems):
    c = lax.axis_index("c")
    s = lax.axis_index("s")
    w = s * 2 + c

    pltpu.sync_copy(zy_h, acca.at[pl.ds(s * RPT, RPT)])
    for b in range(NBUFB):
        pltpu.sync_copy(init_h, augs[b])

    def _start(k, b):
        ch = w + NW * k
        pltpu.make_async_copy(ea_h.at[pl.ds(ch * DE, DE)],
                              eabufs[b], esems[b]).start()
        pltpu.make_async_copy(ei_h.at[1, pl.ds(ch * C, C)],
                              dbufs[b], dsems[b]).start()

    def _finish(b):
        pltpu.make_async_copy(ea_h.at[pl.ds(0, DE)], eabufs[b],
                              esems[b]).wait()
        pltpu.make_async_copy(ei_h.at[1, pl.ds(0, C)], dbufs[b],
                              dsems[b]).wait()
        for q in range(DE):
            for r in range(8):
                augs[b][q * 8 + r, pl.ds(0, 16)] = \
                    eabufs[b][q, pl.ds(r * 16, 16)]
        pltpu.make_async_copy(augs[b], acca.at[dbufs[b]],
                              ssems[b]).start(add=True)

    plsc.subcore_barrier()

    _start(0, 0)

    def _loop(j, carry):
        for b in range(NBUFB):
            k = NBUFB * j + b

            @pl.when(k >= NBUFB - 1)
            def _():
                pltpu.make_async_copy(augs[(b + 1) % NBUFB],
                                      acca.at[dbufs[(b + 1) % NBUFB]],
                                      ssems[(b + 1) % NBUFB]).wait()

            @pl.when(k + 1 < CPT)
            def _():
                _start(k + 1, (b + 1) % NBUFB)

            _finish(b)
        return carry
    lax.fori_loop(0, CPT // NBUFB, _loop, 0)

    for t in range(1, NBUFB):
        pltpu.make_async_copy(augs[t], acca.at[dbufs[t]], ssems[t]).wait()

    @pl.when(w < REM)
    def _():
        _start(CPT, 0)
        _finish(0)
        pltpu.make_async_copy(augs[0], acca.at[dbufs[0]], ssems[0]).wait()

    plsc.subcore_barrier()

    pltpu.sync_copy(acca.at[pl.ds(s * RPT, RPT)],
                    outa_h.at[c, pl.ds(s * RPT, RPT)])


def _sc_easum(ei, eap, init, zy):
    mesh = plsc.VectorSubcoreMesh(core_axis_name="c", subcore_axis_name="s")
    return pl.kernel(
        _scb_body,
        out_type=jax.ShapeDtypeStruct((2, NP, D), jnp.float32),
        mesh=mesh,
        scratch_types=[
            [pltpu.VMEM((C,), jnp.int32) for _ in range(NBUFB)],
            [pltpu.VMEM((DE, C), jnp.float32) for _ in range(NBUFB)],
            [pltpu.VMEM((C, D), jnp.float32) for _ in range(NBUFB)],
            pltpu.VMEM_SHARED((NP, D), jnp.float32),
            [pltpu.SemaphoreType.DMA for _ in range(NBUFB)],
            [pltpu.SemaphoreType.DMA for _ in range(NBUFB)],
            [pltpu.SemaphoreType.DMA for _ in range(NBUFB)],
        ],
    )(ei, eap, init, zy)


def _tc2_body(hs_ref, ay_ref, aa_ref, we_ref, wn_ref, b_ref, o_ref):
    w2 = jnp.dot(we_ref[...], wn_ref[...], preferred_element_type=jnp.float32)
    sy = ay_ref[0] + ay_ref[1]
    aug = aa_ref[0] + aa_ref[1]
    se = aug[:, :DE]
    cnt = aug[:, DE:DE + 1]
    inv = 1.0 / jnp.maximum(cnt, 1.0)
    h = (hs_ref[...]
         + (sy + jnp.dot(se, w2, preferred_element_type=jnp.float32)) * inv
         + b_ref[...])
    o_ref[...] = jnp.where(h > 0, h, jnp.exp(h) - 1.0)


def _tc2(hs, ay, aa, W_edge, W_nbr, b2):
    B = 1000
    return pl.pallas_call(
        _tc2_body,
        grid=(N // B,),
        in_specs=[
            pl.BlockSpec((B, D), lambda i: (i, 0)),
            pl.BlockSpec((2, B, D), lambda i: (0, i, 0)),
            pl.BlockSpec((2, B, D), lambda i: (0, i, 0)),
            pl.BlockSpec((DE, 2 * D), lambda i: (0, 0)),
            pl.BlockSpec((2 * D, D), lambda i: (0, 0)),
            pl.BlockSpec((1, D), lambda i: (0, 0)),
        ],
        out_specs=pl.BlockSpec((B, D), lambda i: (i, 0)),
        out_shape=jax.ShapeDtypeStruct((N, D), jnp.float32),
    )(hs, ay, aa, W_edge, W_nbr, b2)


def kernel(x, z, edge_index, edge_attr, z_table, W_self, W_nbr, W_edge, b):
    z2 = z.astype(jnp.int32).reshape(N, 1)
    ei = edge_index.astype(jnp.int32)
    eap = edge_attr.reshape(E // 8, C)
    ztp = jnp.zeros((D, D), jnp.float32).at[:ZV].set(z_table)

    y = _tc_proj(x, z2, ztp, W_nbr)
    hs = _tc_proj(x, z2, ztp, W_self)

    zy = jnp.zeros((RPT, D), jnp.float32)
    init = jnp.zeros((C, D), jnp.float32).at[:, DE].set(1.0)
    aa = _sc_easum(ei, eap, init, zy)
    ay = _sc_ysum(y, ei, zy)

    return _tc2(hs, ay, aa, W_edge, W_nbr, b.reshape(1, D))


# speedup vs baseline: 10.3167x; 1.0002x over previous
import jax
import jax.numpy as jnp
from jax import lax
from jax.experimental import pallas as pl
from jax.experimental.pallas import tpu as pltpu
from jax.experimental.pallas import tpu_sc as plsc

N = 10000
E = 320000
D = 128
DE = 16
ZV = 100

C = 128
NW = 32
NCHUNK = E // C
CPT = NCHUNK // NW
REM = NCHUNK - CPT * NW
NP = 10112
RPT = NP // 16
NBUFA = 3
NBUFB = 2


def _tcp_body(x_ref, z_ref, zt_ref, w_ref, o_ref, t_ref):
    @pl.when(pl.program_id(0) == 0)
    def _():
        t_ref[...] = jnp.dot(zt_ref[...], w_ref[D:, :],
                             preferred_element_type=jnp.float32)

    oh = (z_ref[...] == lax.broadcasted_iota(jnp.int32, (1, D), 1)
          ).astype(jnp.float32)
    o_ref[...] = (jnp.dot(x_ref[...], w_ref[:D, :],
                          preferred_element_type=jnp.float32)
                  + jnp.dot(oh, t_ref[...],
                            preferred_element_type=jnp.float32))


def _tc_proj(x, z2, ztp, W):
    B = 2000
    return pl.pallas_call(
        _tcp_body,
        grid=(N // B,),
        in_specs=[
            pl.BlockSpec((B, D), lambda i: (i, 0)),
            pl.BlockSpec((B, 1), lambda i: (i, 0)),
            pl.BlockSpec((D, D), lambda i: (0, 0)),
            pl.BlockSpec((2 * D, D), lambda i: (0, 0)),
        ],
        out_specs=pl.BlockSpec((B, D), lambda i: (i, 0)),
        out_shape=jax.ShapeDtypeStruct((N, D), jnp.float32),
        scratch_shapes=[
            pltpu.VMEM((D, D), jnp.float32),
        ],
    )(x, z2, ztp, W)


def _sca_body(y_h, ei_h, zy_h, outy_h,
              sbufs, dbufs, ybufs, accy, gsems, dsems, ssems):
    c = lax.axis_index("c")
    s = lax.axis_index("s")
    w = s * 2 + c

    pltpu.sync_copy(zy_h, accy.at[pl.ds(s * RPT, RPT)])

    def _start(k, b):
        ch = w + NW * k
        pltpu.sync_copy(ei_h.at[0, pl.ds(ch * C, C)], sbufs[b])
        pltpu.make_async_copy(y_h.at[sbufs[b]], ybufs[b], gsems[b]).start()
        pltpu.make_async_copy(ei_h.at[1, pl.ds(ch * C, C)],
                              dbufs[b], dsems[b]).start()

    def _finish(b):
        pltpu.make_async_copy(y_h.at[sbufs[b]], ybufs[b], gsems[b]).wait()
        pltpu.make_async_copy(ei_h.at[1, pl.ds(0, C)], dbufs[b],
                              dsems[b]).wait()
        pltpu.make_async_copy(ybufs[b], accy.at[dbufs[b]],
                              ssems[b]).start(add=True)

    plsc.subcore_barrier()

    _start(0, 0)

    def _loop(j, carry):
        for b in range(NBUFA):
            k = NBUFA * j + b

            @pl.when(k >= NBUFA - 1)
            def _():
                pltpu.make_async_copy(ybufs[(b + 1) % NBUFA],
                                      accy.at[dbufs[(b + 1) % NBUFA]],
                                      ssems[(b + 1) % NBUFA]).wait()

            @pl.when(k + 1 < CPT)
            def _():
                _start(k + 1, (b + 1) % NBUFA)

            _finish(b)
        return carry
    lax.fori_loop(0, CPT // NBUFA, _loop, 0)

    for t in range(1, NBUFA):
        pltpu.make_async_copy(ybufs[t], accy.at[dbufs[t]], ssems[t]).wait()

    @pl.when(w < REM)
    def _():
        _start(CPT, 0)
        _finish(0)
        pltpu.make_async_copy(ybufs[0], accy.at[dbufs[0]], ssems[0]).wait()

    plsc.subcore_barrier()

    pltpu.sync_copy(accy.at[pl.ds(s * RPT, RPT)],
                    outy_h.at[c, pl.ds(s * RPT, RPT)])


def _sc_ysum(y, ei, zy):
    mesh = plsc.VectorSubcoreMesh(core_axis_name="c", subcore_axis_name="s")
    return pl.kernel(
        _sca_body,
        out_type=jax.ShapeDtypeStruct((2, NP, D), jnp.float32),
        mesh=mesh,
        scratch_types=[
            [pltpu.VMEM((C,), jnp.int32) for _ in range(NBUFA)],
            [pltpu.VMEM((C,), jnp.int32) for _ in range(NBUFA)],
            [pltpu.VMEM((C, D), jnp.float32) for _ in range(NBUFA)],
            pltpu.VMEM_SHARED((NP, D), jnp.float32),
            [pltpu.SemaphoreType.DMA for _ in range(NBUFA)],
            [pltpu.SemaphoreType.DMA for _ in range(NBUFA)],
            [pltpu.SemaphoreType.DMA for _ in range(NBUFA)],
        ],
    )(y, ei, zy)


def _scb_body(ei_h, ea_h, init_h, zy_h, outa_h,
              dbufs, eabufs, augs, acca, esems, dsems, ssems):
    c = lax.axis_index("c")
    s = lax.axis_index("s")
    w = s * 2 + c

    pltpu.sync_copy(zy_h, acca.at[pl.ds(s * RPT, RPT)])
    for b in range(NBUFB):
        pltpu.sync_copy(init_h, augs[b])

    def _start(k, b):
        ch = w + NW * k
        pltpu.make_async_copy(ea_h.at[pl.ds(ch * DE, DE)],
                              eabufs[b], esems[b]).start()
        pltpu.make_async_copy(ei_h.at[1, pl.ds(ch * C, C)],
                              dbufs[b], dsems[b]).start()

    def _finish(b):
        pltpu.make_async_copy(ea_h.at[pl.ds(0, DE)], eabufs[b],
                              esems[b]).wait()
        pltpu.make_async_copy(ei_h.at[1, pl.ds(0, C)], dbufs[b],
                              dsems[b]).wait()
        for q in range(DE):
            for r in range(8):
                augs[b][q * 8 + r, pl.ds(0, 16)] = \
                    eabufs[b][q, pl.ds(r * 16, 16)]
        pltpu.make_async_copy(augs[b], acca.at[dbufs[b]],
                              ssems[b]).start(add=True)

    plsc.subcore_barrier()

    _start(0, 0)

    def _loop(j, carry):
        for b in range(NBUFB):
            k = NBUFB * j + b

            @pl.when(k >= NBUFB - 1)
            def _():
                pltpu.make_async_copy(augs[(b + 1) % NBUFB],
                                      acca.at[dbufs[(b + 1) % NBUFB]],
                                      ssems[(b + 1) % NBUFB]).wait()

            @pl.when(k + 1 < CPT)
            def _():
                _start(k + 1, (b + 1) % NBUFB)

            _finish(b)
        return carry
    lax.fori_loop(0, CPT // NBUFB, _loop, 0)

    for t in range(1, NBUFB):
        pltpu.make_async_copy(augs[t], acca.at[dbufs[t]], ssems[t]).wait()

    @pl.when(w < REM)
    def _():
        _start(CPT, 0)
        _finish(0)
        pltpu.make_async_copy(augs[0], acca.at[dbufs[0]], ssems[0]).wait()

    plsc.subcore_barrier()

    pltpu.sync_copy(acca.at[pl.ds(s * RPT, RPT)],
                    outa_h.at[c, pl.ds(s * RPT, RPT)])


def _sc_easum(ei, eap, init, zy):
    mesh = plsc.VectorSubcoreMesh(core_axis_name="c", subcore_axis_name="s")
    return pl.kernel(
        _scb_body,
        out_type=jax.ShapeDtypeStruct((2, NP, D), jnp.float32),
        mesh=mesh,
        scratch_types=[
            [pltpu.VMEM((C,), jnp.int32) for _ in range(NBUFB)],
            [pltpu.VMEM((DE, C), jnp.float32) for _ in range(NBUFB)],
            [pltpu.VMEM((C, D), jnp.float32) for _ in range(NBUFB)],
            pltpu.VMEM_SHARED((NP, D), jnp.float32),
            [pltpu.SemaphoreType.DMA for _ in range(NBUFB)],
            [pltpu.SemaphoreType.DMA for _ in range(NBUFB)],
            [pltpu.SemaphoreType.DMA for _ in range(NBUFB)],
        ],
    )(ei, eap, init, zy)


def _tc2_body(hs_ref, ay_ref, aa_ref, we_ref, wn_ref, b_ref, o_ref):
    w2 = jnp.dot(we_ref[...], wn_ref[...], preferred_element_type=jnp.float32)
    sy = ay_ref[0] + ay_ref[1]
    aug = aa_ref[0] + aa_ref[1]
    se = aug[:, :DE]
    cnt = aug[:, DE:DE + 1]
    inv = 1.0 / jnp.maximum(cnt, 1.0)
    h = (hs_ref[...]
         + (sy + jnp.dot(se, w2, preferred_element_type=jnp.float32)) * inv
         + b_ref[...])
    o_ref[...] = jnp.where(h > 0, h, jnp.exp(h) - 1.0)


def _tc2(hs, ay, aa, W_edge, W_nbr, b2):
    B = 1000
    return pl.pallas_call(
        _tc2_body,
        grid=(N // B,),
        in_specs=[
            pl.BlockSpec((B, D), lambda i: (i, 0)),
            pl.BlockSpec((2, B, D), lambda i: (0, i, 0)),
            pl.BlockSpec((2, B, D), lambda i: (0, i, 0)),
            pl.BlockSpec((DE, 2 * D), lambda i: (0, 0)),
            pl.BlockSpec((2 * D, D), lambda i: (0, 0)),
            pl.BlockSpec((1, D), lambda i: (0, 0)),
        ],
        out_specs=pl.BlockSpec((B, D), lambda i: (i, 0)),
        out_shape=jax.ShapeDtypeStruct((N, D), jnp.float32),
    )(hs, ay, aa, W_edge, W_nbr, b2)


def kernel(x, z, edge_index, edge_attr, z_table, W_self, W_nbr, W_edge, b):
    z2 = z.astype(jnp.int32).reshape(N, 1)
    ei = edge_index.astype(jnp.int32)
    eap = edge_attr.reshape(E // 8, C)
    ztp = jnp.zeros((D, D), jnp.float32).at[:ZV].set(z_table)

    y = _tc_proj(x, z2, ztp, W_nbr)
    hs = _tc_proj(x, z2, ztp, W_self)

    zy = jnp.zeros((RPT, D), jnp.float32)
    init = jnp.zeros((C, D), jnp.float32).at[:, DE].set(1.0)
    aa = _sc_easum(ei, eap, init, zy)
    ay = _sc_ysum(y, ei, zy)

    return _tc2(hs, ay, aa, W_edge, W_nbr, b.reshape(1, D))
